# baseline (speedup 1.0000x reference)
import jax
import jax.numpy as jnp
from jax.experimental import pallas as pl


def _bias_kernel(x_ref, alpha_ref, bmat_ref):
    Bb, ND = x_ref.shape
    N = alpha_ref.shape[1]
    D = ND // N
    x = x_ref[:, :]
    rows = jax.lax.broadcasted_iota(jnp.int32, (ND, N), 0)
    cols = jax.lax.broadcasted_iota(jnp.int32, (ND, N), 1)
    E = jnp.where(rows == D * cols + (D - 1), 1.0, 0.0)
    tid = jax.lax.dot(x, E, preferred_element_type=jnp.float32)
    is_car = jnp.where(tid == 2.0, 1.0, 0.0)
    is_goal = jnp.where(tid == 3.0, 1.0, 0.0)
    is_player = jnp.where(tid == 1.0, 1.0, 0.0)
    alpha_ref[:, :] = 2.0 * (0.6 * is_car + 0.3 * is_goal + 0.1 * is_player) - 1.0
    col = 0.8 * (is_goal - is_car)
    bmat_ref[:, :, :] = is_player[:, :, None] * col[:, None, :]


def kernel(obj_slots):
    Bsz, N, D = obj_slots.shape
    x = obj_slots.reshape(Bsz, N * D)
    Bb = 256
    grid = (Bsz // Bb,)
    alpha, bmat = pl.pallas_call(
        _bias_kernel,
        grid=grid,
        in_specs=[pl.BlockSpec((Bb, N * D), lambda i: (i, 0))],
        out_specs=[
            pl.BlockSpec((Bb, N), lambda i: (i, 0)),
            pl.BlockSpec((Bb, N, N), lambda i: (i, 0, 0)),
        ],
        out_shape=[
            jax.ShapeDtypeStruct((Bsz, N), jnp.float32),
            jax.ShapeDtypeStruct((Bsz, N, N), jnp.float32),
        ],
    )(x)
    return (alpha, bmat)


# speedup vs baseline: 1.4255x; 1.4255x over previous
import jax
import jax.numpy as jnp
from jax.experimental import pallas as pl


def _bias_kernel(x_ref, alpha_ref, bmat_ref):
    Bb, ND = x_ref.shape
    N = alpha_ref.shape[1]
    D = ND // N
    x = x_ref[:, :]
    rows = jax.lax.broadcasted_iota(jnp.int32, (ND, N), 0)
    cols = jax.lax.broadcasted_iota(jnp.int32, (ND, N), 1)
    E = jnp.where(rows == D * cols + (D - 1), 1.0, 0.0)
    tid = jax.lax.dot(x, E, preferred_element_type=jnp.float32)
    is_car = jnp.where(tid == 2.0, 1.0, 0.0)
    is_goal = jnp.where(tid == 3.0, 1.0, 0.0)
    is_player = jnp.where(tid == 1.0, 1.0, 0.0)
    alpha_ref[:, :] = 2.0 * (0.6 * is_car + 0.3 * is_goal + 0.1 * is_player) - 1.0
    col = 0.8 * (is_goal - is_car)
    col2 = jnp.concatenate([col, col], axis=1)
    lane = jax.lax.broadcasted_iota(jnp.int32, (Bb, 2 * N), 1)
    in_lo = lane < N
    for i in range(N // 2):
        lo = jax.lax.broadcast_in_dim(
            is_player[:, 2 * i : 2 * i + 1], (Bb, 2 * N), (0, 1)
        )
        hi = jax.lax.broadcast_in_dim(
            is_player[:, 2 * i + 1 : 2 * i + 2], (Bb, 2 * N), (0, 1)
        )
        pr = jnp.where(in_lo, lo, hi)
        bmat_ref[:, 2 * N * i : 2 * N * (i + 1)] = pr * col2


def kernel(obj_slots):
    Bsz, N, D = obj_slots.shape
    x = obj_slots.reshape(Bsz, N * D)
    Bb = 256
    grid = (Bsz // Bb,)
    alpha, bmat = pl.pallas_call(
        _bias_kernel,
        grid=grid,
        in_specs=[pl.BlockSpec((Bb, N * D), lambda i: (i, 0))],
        out_specs=[
            pl.BlockSpec((Bb, N), lambda i: (i, 0)),
            pl.BlockSpec((Bb, N * N), lambda i: (i, 0)),
        ],
        out_shape=[
            jax.ShapeDtypeStruct((Bsz, N), jnp.float32),
            jax.ShapeDtypeStruct((Bsz, N * N), jnp.float32),
        ],
    )(x)
    return (alpha, bmat.reshape(Bsz, N, N))
